# baseline (speedup 1.0000x reference)
import functools

import jax
import jax.numpy as jnp
from jax import lax
from jax.experimental import pallas as pl
from jax.experimental.pallas import tpu as pltpu
from jax.experimental.pallas import tpu_sc as plsc


@functools.cache
def _sc_row_copy(n_elems: int):
    info = plsc.get_sparse_core_info()
    num_workers = info.num_cores * info.num_subcores
    assert n_elems % num_workers == 0
    per_w = n_elems // num_workers
    assert per_w % 8 == 0

    mesh = plsc.VectorSubcoreMesh(core_axis_name="c", subcore_axis_name="s")

    @functools.partial(
        pl.kernel,
        out_type=jax.ShapeDtypeStruct((n_elems,), jnp.float32),
        mesh=mesh,
        scratch_types=[pltpu.VMEM((per_w,), jnp.float32)],
    )
    def copy_kernel(tab_hbm, out_hbm, buf):
        wid = lax.axis_index("s") * info.num_cores + lax.axis_index("c")
        base = wid * per_w
        pltpu.sync_copy(tab_hbm.at[pl.ds(base, per_w)], buf)
        pltpu.sync_copy(buf, out_hbm.at[pl.ds(base, per_w)])

    return copy_kernel


def kernel(x, wpe):
    seq_len = x.shape[1]
    emb_dim = wpe.shape[1]
    flat = jnp.reshape(wpe, (-1,))
    out = _sc_row_copy(seq_len * emb_dim)(flat)
    return jnp.reshape(out, (1, seq_len, emb_dim))
